# baseline (speedup 1.0000x reference)
import functools

import jax
import jax.numpy as jnp
from jax import lax
from jax.experimental import pallas as pl
from jax.experimental.pallas import tpu as pltpu
from jax.experimental.pallas import tpu_sc as plsc

V_MIN = -10.0
V_MAX = 10.0
NUM_ATOMS = 51
DISCOUNT = 0.99
ATOM_DELTA = (V_MAX - V_MIN) / (NUM_ATOMS - 1)
INV_DELTA = 2.5

NC = 2
NS = 16
NW = NC * NS
LANES = 16


def _sc_body(rows_w, rew_hbm, nd_hbm, probs_hbm, out_hbm,
             rew_v, nd_v, probs_v, out_v):
    wid = lax.axis_index("s") * NC + lax.axis_index("c")
    base = wid * rows_w

    pltpu.sync_copy(rew_hbm.at[pl.ds(base, rows_w)], rew_v)
    pltpu.sync_copy(nd_hbm.at[pl.ds(base, rows_w)], nd_v)
    pltpu.sync_copy(probs_hbm.at[pl.ds(base, rows_w)], probs_v)

    iota = lax.iota(jnp.int32, LANES)
    zeros_i = jnp.zeros((LANES,), jnp.int32)
    zeros16 = jnp.zeros((LANES,), jnp.float32)
    nblocks = rows_w // LANES

    def block(b, _):
        rvec = iota + b * LANES
        rew = plsc.load_gather(rew_v, [rvec, zeros_i])
        g = plsc.load_gather(nd_v, [rvec, zeros_i]) * DISCOUNT
        for k in range(NUM_ATOMS):
            plsc.store_scatter(out_v, [rvec, zeros_i + k], zeros16)
        for j in range(NUM_ATOMS):
            a_j = V_MIN + ATOM_DELTA * j
            p = plsc.load_gather(probs_v, [rvec, zeros_i + j])
            val = rew + g * a_j
            val = jnp.minimum(jnp.maximum(val, V_MIN), V_MAX)
            xf = (val - V_MIN) * INV_DELTA
            li = xf.astype(jnp.int32)
            frac = xf - li.astype(jnp.float32)
            uv = frac * p
            lv = p - uv
            ui = jnp.minimum(li + 1, NUM_ATOMS - 1)
            plsc.addupdate_scatter(out_v, [rvec, li], lv)
            plsc.addupdate_scatter(out_v, [rvec, ui], uv)
        return _

    lax.fori_loop(0, nblocks, block, None)
    pltpu.sync_copy(out_v, out_hbm.at[pl.ds(base, rows_w)])


@jax.jit
def kernel(reward, probs, not_done):
    bs = probs.shape[0]
    rows_w = bs // NW
    mesh = plsc.VectorSubcoreMesh(
        core_axis_name="c", subcore_axis_name="s",
        num_cores=NC, num_subcores=NS)
    run = pl.kernel(
        functools.partial(_sc_body, rows_w),
        out_type=jax.ShapeDtypeStruct((bs, NUM_ATOMS), jnp.float32),
        mesh=mesh,
        compiler_params=pltpu.CompilerParams(
            needs_layout_passes=False, use_tc_tiling_on_sc=False),
        scratch_types=[
            pltpu.VMEM((rows_w, 1), jnp.float32),
            pltpu.VMEM((rows_w, 1), jnp.float32),
            pltpu.VMEM((rows_w, NUM_ATOMS), jnp.float32),
            pltpu.VMEM((rows_w, NUM_ATOMS), jnp.float32),
        ],
    )
    return run(reward, not_done, probs)


# speedup vs baseline: 1.1359x; 1.1359x over previous
import functools

import jax
import jax.numpy as jnp
from jax import lax
from jax.experimental import pallas as pl
from jax.experimental.pallas import tpu as pltpu
from jax.experimental.pallas import tpu_sc as plsc

V_MIN = -10.0
V_MAX = 10.0
NUM_ATOMS = 51
DISCOUNT = 0.99
ATOM_DELTA = (V_MAX - V_MIN) / (NUM_ATOMS - 1)
INV_DELTA = 2.5

NC = 2
NS = 16
NW = NC * NS
LANES = 16
CHUNK = 128


def _sc_body(rows_w, rew_hbm, nd_hbm, probs_hbm, out_hbm,
             rew_v, nd_v, probs_v, out_v):
    wid = lax.axis_index("s") * NC + lax.axis_index("c")
    base = wid * rows_w
    nchunks = rows_w // CHUNK

    iota = lax.iota(jnp.int32, LANES)
    zeros_i = jnp.zeros((LANES,), jnp.int32)
    zeros16 = jnp.zeros((LANES,), jnp.float32)
    nblocks = CHUNK // LANES

    def chunk_step(c, _):
        cbase = base + c * CHUNK
        pltpu.sync_copy(rew_hbm.at[pl.ds(cbase, CHUNK)], rew_v)
        pltpu.sync_copy(nd_hbm.at[pl.ds(cbase, CHUNK)], nd_v)
        pltpu.sync_copy(probs_hbm.at[pl.ds(cbase, CHUNK)], probs_v)

        def block(b, _):
            rvec = iota + b * LANES
            rew = plsc.load_gather(rew_v, [rvec, zeros_i])
            g = plsc.load_gather(nd_v, [rvec, zeros_i]) * DISCOUNT
            for k in range(NUM_ATOMS):
                plsc.store_scatter(out_v, [rvec, zeros_i + k], zeros16)
            for j in range(NUM_ATOMS):
                a_j = V_MIN + ATOM_DELTA * j
                p = plsc.load_gather(probs_v, [rvec, zeros_i + j])
                val = rew + g * a_j
                val = jnp.minimum(jnp.maximum(val, V_MIN), V_MAX)
                xf = (val - V_MIN) * INV_DELTA
                li = xf.astype(jnp.int32)
                frac = xf - li.astype(jnp.float32)
                uv = frac * p
                lv = p - uv
                ui = jnp.minimum(li + 1, NUM_ATOMS - 1)
                plsc.addupdate_scatter(out_v, [rvec, li], lv)
                plsc.addupdate_scatter(out_v, [rvec, ui], uv)
            return _

        lax.fori_loop(0, nblocks, block, None)
        pltpu.sync_copy(out_v, out_hbm.at[pl.ds(cbase, CHUNK)])
        return _

    lax.fori_loop(0, nchunks, chunk_step, None)


@jax.jit
def kernel(reward, probs, not_done):
    bs = probs.shape[0]
    rows_w = bs // NW
    mesh = plsc.VectorSubcoreMesh(
        core_axis_name="c", subcore_axis_name="s",
        num_cores=NC, num_subcores=NS)
    run = pl.kernel(
        functools.partial(_sc_body, rows_w),
        out_type=jax.ShapeDtypeStruct((bs, NUM_ATOMS), jnp.float32),
        mesh=mesh,
        compiler_params=pltpu.CompilerParams(needs_layout_passes=False),
        scratch_types=[
            pltpu.VMEM((CHUNK, 1), jnp.float32),
            pltpu.VMEM((CHUNK, 1), jnp.float32),
            pltpu.VMEM((CHUNK, NUM_ATOMS), jnp.float32),
            pltpu.VMEM((CHUNK, NUM_ATOMS), jnp.float32),
        ],
    )
    return run(reward, not_done, probs)


# speedup vs baseline: 1.4850x; 1.3073x over previous
import functools

import jax
import jax.numpy as jnp
from jax import lax
from jax.experimental import pallas as pl
from jax.experimental.pallas import tpu as pltpu
from jax.experimental.pallas import tpu_sc as plsc

V_MIN = -10.0
V_MAX = 10.0
NUM_ATOMS = 51
DISCOUNT = 0.99
ATOM_DELTA = (V_MAX - V_MIN) / (NUM_ATOMS - 1)
INV_DELTA = 2.5

NC = 2
NS = 16
NW = NC * NS
LANES = 16
CHUNK = 128


def _sc_body(rows_w, rew_hbm, nd_hbm, probs_hbm, out_hbm,
             rew_v, nd_v, pc0, pc1, oc0, oc1,
             psem0, psem1, osem0, osem1, rsem):
    wid = lax.axis_index("s") * NC + lax.axis_index("c")
    base = wid * rows_w
    nch = rows_w // CHUNK
    pbufs, obufs = [pc0, pc1], [oc0, oc1]
    psems, osems = [psem0, psem1], [osem0, osem1]

    iota = lax.iota(jnp.int32, LANES)
    zeros_i = jnp.zeros((LANES,), jnp.int32)
    zeros16 = jnp.zeros((LANES,), jnp.float32)
    nblocks = CHUNK // LANES

    rdma = pltpu.async_copy(rew_hbm.at[pl.ds(base, rows_w)], rew_v, rsem)
    ndma = pltpu.async_copy(nd_hbm.at[pl.ds(base, rows_w)], nd_v, rsem)
    pdma = [None] * nch
    odma = [None] * nch
    pdma[0] = pltpu.async_copy(
        probs_hbm.at[pl.ds(base, CHUNK)], pbufs[0], psems[0])
    rdma.wait()
    ndma.wait()

    for c in range(nch):
        if c + 1 < nch:
            pdma[c + 1] = pltpu.async_copy(
                probs_hbm.at[pl.ds(base + (c + 1) * CHUNK, CHUNK)],
                pbufs[(c + 1) % 2], psems[(c + 1) % 2])
        pdma[c].wait()
        if c >= 2:
            odma[c - 2].wait()
        pbuf, obuf = pbufs[c % 2], obufs[c % 2]
        cb = c * CHUNK

        @plsc.parallel_loop(0, nblocks)
        def block(b):
            rvec = iota + b * LANES
            rew = rew_v[pl.ds(cb + b * LANES, LANES)]
            g = nd_v[pl.ds(cb + b * LANES, LANES)] * DISCOUNT
            for k in range(NUM_ATOMS):
                plsc.store_scatter(obuf, [rvec, zeros_i + k], zeros16)
            for j in range(NUM_ATOMS):
                a_j = V_MIN + ATOM_DELTA * j
                p = plsc.load_gather(pbuf, [rvec, zeros_i + j])
                val = rew + g * a_j
                val = jnp.minimum(jnp.maximum(val, V_MIN), V_MAX)
                xf = (val - V_MIN) * INV_DELTA
                li = xf.astype(jnp.int32)
                frac = xf - li.astype(jnp.float32)
                uv = frac * p
                lv = p - uv
                ui = jnp.minimum(li + 1, NUM_ATOMS - 1)
                plsc.addupdate_scatter(obuf, [rvec, li], lv)
                plsc.addupdate_scatter(obuf, [rvec, ui], uv)

        odma[c] = pltpu.async_copy(
            obuf, out_hbm.at[pl.ds(base + c * CHUNK, CHUNK)], osems[c % 2])

    odma[nch - 2].wait()
    odma[nch - 1].wait()


@jax.jit
def kernel(reward, probs, not_done):
    bs = probs.shape[0]
    rows_w = bs // NW
    mesh = plsc.VectorSubcoreMesh(
        core_axis_name="c", subcore_axis_name="s",
        num_cores=NC, num_subcores=NS)
    run = pl.kernel(
        functools.partial(_sc_body, rows_w),
        out_type=jax.ShapeDtypeStruct((bs, NUM_ATOMS), jnp.float32),
        mesh=mesh,
        compiler_params=pltpu.CompilerParams(needs_layout_passes=False),
        scratch_types=[
            pltpu.VMEM((rows_w,), jnp.float32),
            pltpu.VMEM((rows_w,), jnp.float32),
            pltpu.VMEM((CHUNK, NUM_ATOMS), jnp.float32),
            pltpu.VMEM((CHUNK, NUM_ATOMS), jnp.float32),
            pltpu.VMEM((CHUNK, NUM_ATOMS), jnp.float32),
            pltpu.VMEM((CHUNK, NUM_ATOMS), jnp.float32),
            pltpu.SemaphoreType.DMA,
            pltpu.SemaphoreType.DMA,
            pltpu.SemaphoreType.DMA,
            pltpu.SemaphoreType.DMA,
            pltpu.SemaphoreType.DMA,
        ],
    )
    return run(reward.reshape(-1), not_done.reshape(-1), probs)
